# Initial kernel scaffold; baseline (speedup 1.0000x reference)
import jax
import jax.numpy as jnp
from jax.experimental import pallas as pl


def kernel(x, pe):
    raise NotImplementedError("write your pallas kernel here")


# speedup vs baseline: 1.6698x; 1.6698x over previous
import jax
import jax.numpy as jnp
from jax.experimental import pallas as pl

_BT = 256


def _add_kernel(x_ref, pe_ref, o_ref):
    o_ref[...] = x_ref[...] + pe_ref[...]


def kernel(x, pe):
    B, T, D = x.shape
    bt = _BT if T % _BT == 0 else T
    grid = (T // bt, B)
    return pl.pallas_call(
        _add_kernel,
        grid=grid,
        in_specs=[
            pl.BlockSpec((1, bt, D), lambda i, b: (b, i, 0)),
            pl.BlockSpec((bt, D), lambda i, b: (i, 0)),
        ],
        out_specs=pl.BlockSpec((1, bt, D), lambda i, b: (b, i, 0)),
        out_shape=jax.ShapeDtypeStruct(x.shape, x.dtype),
    )(x, pe[:T])


# speedup vs baseline: 1.7409x; 1.0426x over previous
import jax
import jax.numpy as jnp
from jax.experimental import pallas as pl

_BT = 512


def _add_kernel(x_ref, pe_ref, o_ref):
    o_ref[...] = x_ref[...] + pe_ref[...]


def kernel(x, pe):
    B, T, D = x.shape
    bt = _BT if T % _BT == 0 else T
    grid = (T // bt, B)
    return pl.pallas_call(
        _add_kernel,
        grid=grid,
        in_specs=[
            pl.BlockSpec((1, bt, D), lambda i, b: (b, i, 0)),
            pl.BlockSpec((bt, D), lambda i, b: (i, 0)),
        ],
        out_specs=pl.BlockSpec((1, bt, D), lambda i, b: (b, i, 0)),
        out_shape=jax.ShapeDtypeStruct(x.shape, x.dtype),
    )(x, pe[:T])
